# baseline (speedup 1.0000x reference)
import jax
import jax.numpy as jnp
from jax.experimental import pallas as pl
from jax.experimental.pallas import tpu as pltpu


_B = 128
_N = 1000001
_BLOCK_N = 8192
_NBUF = 4
_GRID = _N // _BLOCK_N
_LAST = _GRID - 1
_TAIL_BN = 2048
_TAIL_IDX = _GRID * _BLOCK_N // _TAIL_BN


def _copy_out(scratch, o_ref, sem, step):
    return pltpu.make_async_copy(
        scratch,
        o_ref.at[:, pl.ds(step * _BLOCK_N, _BLOCK_N)],
        sem,
    )


def _bulk_kernel(x_ref, w_ref, b_ref, o_ref, *scr):
    bufs = scr[:_NBUF]
    sems = scr[_NBUF]
    i = pl.program_id(0)
    slot = jax.lax.rem(i, _NBUF)

    acc = jax.lax.dot_general(
        x_ref[...], w_ref[...], (((1,), (1,)), ((), ())),
        preferred_element_type=jnp.float32,
    ) + b_ref[...]

    for s in range(_NBUF):
        @pl.when(slot == s)
        def _():
            @pl.when(i >= _NBUF)
            def _():
                _copy_out(bufs[s], o_ref, sems.at[s], i - _NBUF).wait()

            bufs[s][...] = acc
            _copy_out(bufs[s], o_ref, sems.at[s], i).start()

    @pl.when(i == _LAST)
    def _():
        for t in range(_LAST - _NBUF + 1, _LAST + 1):
            _copy_out(bufs[t % _NBUF], o_ref, sems.at[t % _NBUF], t).wait()


def _tail_kernel(x_ref, w_ref, b_ref, bulk_ref, o_ref):
    del bulk_ref
    o_ref[...] = jax.lax.dot_general(
        x_ref[...], w_ref[...], (((1,), (1,)), ((), ())),
        preferred_element_type=jnp.float32,
    ) + b_ref[...]


def kernel(x, y, freeze_flag, W, b):
    del y, freeze_flag
    B, D = x.shape
    N = W.shape[0]
    b2 = b.reshape(1, N)
    bulk = pl.pallas_call(
        _bulk_kernel,
        grid=(_GRID,),
        in_specs=[
            pl.BlockSpec((B, D), lambda i: (0, 0)),
            pl.BlockSpec((_BLOCK_N, D), lambda i: (i, 0)),
            pl.BlockSpec((1, _BLOCK_N), lambda i: (0, i)),
        ],
        out_specs=pl.BlockSpec(memory_space=pl.ANY),
        out_shape=jax.ShapeDtypeStruct((B, N), jnp.float32),
        scratch_shapes=(
            [pltpu.VMEM((_B, _BLOCK_N), jnp.float32) for _ in range(_NBUF)]
            + [pltpu.SemaphoreType.DMA((_NBUF,))]
        ),
        compiler_params=pltpu.CompilerParams(
            dimension_semantics=("arbitrary",),
        ),
    )(x, W, b2)
    out = pl.pallas_call(
        _tail_kernel,
        grid=(1,),
        in_specs=[
            pl.BlockSpec((B, D), lambda i: (0, 0)),
            pl.BlockSpec((_TAIL_BN, D), lambda i: (_TAIL_IDX, 0)),
            pl.BlockSpec((1, _TAIL_BN), lambda i: (0, _TAIL_IDX)),
            pl.BlockSpec(memory_space=pl.ANY),
        ],
        out_specs=pl.BlockSpec((B, _TAIL_BN), lambda i: (0, _TAIL_IDX)),
        out_shape=jax.ShapeDtypeStruct((B, N), jnp.float32),
        input_output_aliases={3: 0},
    )(x, W, b2, bulk)
    return out
